# Initial kernel scaffold; baseline (speedup 1.0000x reference)
import jax
import jax.numpy as jnp
from jax.experimental import pallas as pl


def kernel(x):
    raise NotImplementedError("write your pallas kernel here")


# speedup vs baseline: 26.0029x; 26.0029x over previous
import functools

import jax
import jax.numpy as jnp
from jax.experimental import pallas as pl

_K = 2048
_ROWS = 128
_COLS = 32768
_BLOCK_ROWS = 16


def _body(x_ref, o_ref):
    xv = x_ref[...]
    s = jax.lax.bitcast_convert_type(xv, jnp.int32)
    key = jnp.where(s < 0, s ^ jnp.int32(0x7FFFFFFF), s)

    lo0 = jnp.full((_BLOCK_ROWS, 1), jnp.iinfo(jnp.int32).min, jnp.int32)
    hi0 = jnp.full((_BLOCK_ROWS, 1), jnp.iinfo(jnp.int32).max, jnp.int32)

    def step(_, carry):
        lo, hi = carry
        mid = (lo & hi) + ((lo ^ hi) >> 1)
        cnt = jnp.sum((key >= mid).astype(jnp.int32), axis=1, keepdims=True)
        ge = cnt >= _K
        lo = jnp.where(ge, mid, lo)
        hi = jnp.where(ge, hi, mid)
        return lo, hi

    lo, _ = jax.lax.fori_loop(0, 32, step, (lo0, hi0))
    mask = key >= lo
    o_ref[...] = jnp.where(mask, jnp.maximum(xv, 0.0), 0.0)


@jax.jit
def kernel(x):
    grid = (_ROWS // _BLOCK_ROWS,)
    return pl.pallas_call(
        _body,
        grid=grid,
        in_specs=[pl.BlockSpec((_BLOCK_ROWS, _COLS), lambda i: (i, 0))],
        out_specs=pl.BlockSpec((_BLOCK_ROWS, _COLS), lambda i: (i, 0)),
        out_shape=jax.ShapeDtypeStruct((_ROWS, _COLS), jnp.float32),
    )(x)
